# baseline (speedup 1.0000x reference)
import jax
import jax.numpy as jnp
from jax import lax
from jax.experimental import pallas as pl
from jax.experimental.pallas import tpu as pltpu
from jax.experimental.pallas import tpu_sc as plsc

_ACCELERATION = 4.0
_CENTER_FRACTION = 0.08
_W = 320
_SLOPE = 10.0
_NUM_ACTIONS = _W
_BUDGET = int(_NUM_ACTIONS / _ACCELERATION - _NUM_ACTIONS * _CENTER_FRACTION)


def _policy_kernel(mask2d_ref, sampler_ref, u_ref, mpm_ref, rowval_ref):
    m = mask2d_ref[...]
    s = sampler_ref[...]
    u = u_ref[...]
    b = m.shape[0]
    prob = jnp.logaddexp(0.0, _SLOPE * s) / _SLOPE
    prob = jnp.broadcast_to(prob, (b, _W))
    denom = jnp.max((1.0 - m) * prob, axis=1, keepdims=True)
    prob = prob / denom
    mpm = prob * (1.0 - m)
    sel = m == 0
    x = jnp.where(sel, mpm, 0.0)
    sparsity = _BUDGET / _W
    xbar = jnp.mean(x, axis=1, keepdims=True)
    r = sparsity / xbar
    beta = (1.0 - sparsity) / (1.0 - xbar)
    le = (r <= 1.0).astype(x.dtype)
    normed = le * x * r + (1.0 - le) * (1.0 - (1.0 - x) * beta)
    mpm = jnp.where(sel, normed, mpm)
    hard = (mpm > u).astype(mpm.dtype)
    mpm_ref[...] = mpm
    rowval_ref[...] = m + hard


def _mask_kernel(ks_ref, rv_ref, mpm_ref,
                 out_ref, mask_new_ref, fpm_ref):
    c = pl.program_id(1)
    out_ref[...] = ks_ref[...] * rv_ref[...]

    @pl.when(c == 0)
    def _():
        mask_new_ref[...] = jnp.broadcast_to(rv_ref[...], mask_new_ref.shape)
        fpm_ref[...] = jnp.broadcast_to(mpm_ref[...], fpm_ref.shape)


def _sc_mask_copy(mask_hbm, out_hbm, buf):
    wid = lax.axis_index("s") * 2 + lax.axis_index("c")
    b = wid // 16
    rem = wid % 16
    sl = rem // 2
    h0 = (rem % 2) * 160
    src = mask_hbm.at[b, 0, sl, pl.ds(h0, 160)]
    dst = out_hbm.at[b, 0, sl, pl.ds(h0, 160)]
    pltpu.sync_copy(src, buf)
    pltpu.sync_copy(buf, dst)


def kernel(mask, kspace, sampler):
    batch, coils, slc, height, width, _ = kspace.shape
    mask2d = mask[:, :, 0, 0, :, :].reshape(batch, width)
    u = jax.random.uniform(jax.random.key(1), (batch, width), dtype=kspace.dtype)

    mpm, rowval = pl.pallas_call(
        _policy_kernel,
        out_shape=[
            jax.ShapeDtypeStruct((batch, width), kspace.dtype),
            jax.ShapeDtypeStruct((batch, width), kspace.dtype),
        ],
    )(mask2d, sampler, u)

    ksv = jnp.swapaxes(kspace, 4, 5)
    maskv = jnp.swapaxes(mask, 4, 5)
    rv6 = rowval.reshape(batch, 1, 1, 1, 1, width)
    mpm6 = mpm.reshape(batch, 1, 1, 1, 1, width)
    grid = (batch, coils)
    row_spec = pl.BlockSpec((1, 1, 1, 1, 1, width), lambda b, c: (b, 0, 0, 0, 0, 0))
    big_spec = pl.BlockSpec((1, 1, slc, height, 2, width), lambda b, c: (b, c, 0, 0, 0, 0))
    bcast_spec = pl.BlockSpec((1, 1, 1, height, 1, width), lambda b, c: (b, 0, 0, 0, 0, 0))
    out6, mask_new6, fpm6 = pl.pallas_call(
        _mask_kernel,
        grid=grid,
        in_specs=[big_spec, row_spec, row_spec],
        out_specs=[big_spec, bcast_spec, bcast_spec],
        out_shape=[
            jax.ShapeDtypeStruct((batch, coils, slc, height, 2, width), kspace.dtype),
            jax.ShapeDtypeStruct((batch, 1, 1, height, 1, width), kspace.dtype),
            jax.ShapeDtypeStruct((batch, 1, 1, height, 1, width), kspace.dtype),
        ],
    )(ksv, rv6, mpm6)

    mask_o = pl.kernel(
        _sc_mask_copy,
        out_type=jax.ShapeDtypeStruct((batch, 1, slc, height, 1, width), mask.dtype),
        mesh=plsc.VectorSubcoreMesh(core_axis_name="c", subcore_axis_name="s"),
        scratch_types=[
            pltpu.VMEM((160, 1, width), mask.dtype),
        ],
        compiler_params=pltpu.CompilerParams(use_tc_tiling_on_sc=True),
    )(maskv)

    masked_kspace = jnp.swapaxes(out6, 4, 5)
    mask_in = jnp.swapaxes(mask_o, 4, 5)
    mask_new = jnp.swapaxes(mask_new6, 4, 5)
    final_prob_mask = jnp.swapaxes(fpm6, 4, 5)
    return (masked_kspace, mask_in, mask_new, final_prob_mask)


# speedup vs baseline: 1.0008x; 1.0008x over previous
import jax
import jax.numpy as jnp
from jax import lax
from jax.experimental import pallas as pl
from jax.experimental.pallas import tpu as pltpu
from jax.experimental.pallas import tpu_sc as plsc

_ACCELERATION = 4.0
_CENTER_FRACTION = 0.08
_W = 320
_SLOPE = 10.0
_NUM_ACTIONS = _W
_BUDGET = int(_NUM_ACTIONS / _ACCELERATION - _NUM_ACTIONS * _CENTER_FRACTION)


def _policy_kernel(mask2d_ref, sampler_ref, u_ref, mpm_ref, rowval_ref):
    m = mask2d_ref[...]
    s = sampler_ref[...]
    u = u_ref[...]
    b = m.shape[0]
    prob = jnp.logaddexp(0.0, _SLOPE * s) / _SLOPE
    prob = jnp.broadcast_to(prob, (b, _W))
    denom = jnp.max((1.0 - m) * prob, axis=1, keepdims=True)
    prob = prob / denom
    mpm = prob * (1.0 - m)
    sel = m == 0
    x = jnp.where(sel, mpm, 0.0)
    sparsity = _BUDGET / _W
    xbar = jnp.mean(x, axis=1, keepdims=True)
    r = sparsity / xbar
    beta = (1.0 - sparsity) / (1.0 - xbar)
    le = (r <= 1.0).astype(x.dtype)
    normed = le * x * r + (1.0 - le) * (1.0 - (1.0 - x) * beta)
    mpm = jnp.where(sel, normed, mpm)
    hard = (mpm > u).astype(mpm.dtype)
    mpm_ref[...] = mpm
    rowval_ref[...] = m + hard


def _mask_kernel(ks_ref, rv_ref, mpm_ref,
                 out_ref, mask_new_ref, fpm_ref):
    c = pl.program_id(1)
    out_ref[...] = ks_ref[...] * rv_ref[...]

    @pl.when(c == 0)
    def _():
        mask_new_ref[...] = jnp.broadcast_to(rv_ref[...], mask_new_ref.shape)
        fpm_ref[...] = jnp.broadcast_to(mpm_ref[...], fpm_ref.shape)


def _sc_mask_copy(mask_hbm, out_hbm, buf):
    wid = lax.axis_index("s") * 2 + lax.axis_index("c")
    b = wid // 16
    rem = wid % 16
    sl = rem // 2
    h0 = (rem % 2) * 160
    src = mask_hbm.at[b, 0, sl, pl.ds(h0, 160)]
    dst = out_hbm.at[b, 0, sl, pl.ds(h0, 160)]
    pltpu.sync_copy(src, buf)
    pltpu.sync_copy(buf, dst)


def kernel(mask, kspace, sampler):
    batch, coils, slc, height, width, _ = kspace.shape
    mask2d = mask[:, :, 0, 0, :, :].reshape(batch, width)
    u = jax.random.uniform(jax.random.key(1), (batch, width), dtype=kspace.dtype)

    mpm, rowval = pl.pallas_call(
        _policy_kernel,
        out_shape=[
            jax.ShapeDtypeStruct((batch, width), kspace.dtype),
            jax.ShapeDtypeStruct((batch, width), kspace.dtype),
        ],
    )(mask2d, sampler, u)

    ksv = jnp.swapaxes(kspace, 4, 5)
    maskv = jnp.swapaxes(mask, 4, 5)
    rv6 = rowval.reshape(batch, 1, 1, 1, 1, width)
    mpm6 = mpm.reshape(batch, 1, 1, 1, 1, width)
    grid = (batch, coils)
    row_spec = pl.BlockSpec((1, 1, 1, 1, 1, width), lambda b, c: (b, 0, 0, 0, 0, 0))
    big_spec = pl.BlockSpec((1, 1, slc, height, 2, width), lambda b, c: (b, c, 0, 0, 0, 0))
    bcast_spec = pl.BlockSpec((1, 1, 1, height, 1, width), lambda b, c: (b, 0, 0, 0, 0, 0))
    out6, mask_new6, fpm6 = pl.pallas_call(
        _mask_kernel,
        grid=grid,
        in_specs=[big_spec, row_spec, row_spec],
        out_specs=[big_spec, bcast_spec, bcast_spec],
        out_shape=[
            jax.ShapeDtypeStruct((batch, coils, slc, height, 2, width), kspace.dtype),
            jax.ShapeDtypeStruct((batch, 1, 1, height, 1, width), kspace.dtype),
            jax.ShapeDtypeStruct((batch, 1, 1, height, 1, width), kspace.dtype),
        ],
        compiler_params=pltpu.CompilerParams(skip_device_barrier=True),
    )(ksv, rv6, mpm6)

    mask_o = pl.kernel(
        _sc_mask_copy,
        out_type=jax.ShapeDtypeStruct((batch, 1, slc, height, 1, width), mask.dtype),
        mesh=plsc.VectorSubcoreMesh(core_axis_name="c", subcore_axis_name="s"),
        scratch_types=[
            pltpu.VMEM((160, 1, width), mask.dtype),
        ],
        compiler_params=pltpu.CompilerParams(use_tc_tiling_on_sc=True),
    )(maskv)

    masked_kspace = jnp.swapaxes(out6, 4, 5)
    mask_in = jnp.swapaxes(mask_o, 4, 5)
    mask_new = jnp.swapaxes(mask_new6, 4, 5)
    final_prob_mask = jnp.swapaxes(fpm6, 4, 5)
    return (masked_kspace, mask_in, mask_new, final_prob_mask)


# speedup vs baseline: 1.1529x; 1.1520x over previous
import jax
import jax.numpy as jnp
import numpy as np
from jax.experimental import pallas as pl

_ACCELERATION = 4.0
_CENTER_FRACTION = 0.08
_W = 320
_SLOPE = 10.0
_NUM_ACTIONS = _W
_BUDGET = int(_NUM_ACTIONS / _ACCELERATION - _NUM_ACTIONS * _CENTER_FRACTION)

_U_CACHE = {}


def _u_const(shape, dtype):
    k = (shape, np.dtype(dtype).name)
    if k not in _U_CACHE:
        with jax.ensure_compile_time_eval():
            _U_CACHE[k] = np.asarray(
                jax.random.uniform(jax.random.key(1), shape, dtype=dtype))
    return _U_CACHE[k]


def _policy_kernel(mask2d_ref, sampler_ref, u_ref, mpm_ref, rowval_ref):
    m = mask2d_ref[...]
    s = sampler_ref[...]
    u = u_ref[...]
    b = m.shape[0]
    prob = jnp.logaddexp(0.0, _SLOPE * s) / _SLOPE
    prob = jnp.broadcast_to(prob, (b, _W))
    denom = jnp.max((1.0 - m) * prob, axis=1, keepdims=True)
    prob = prob / denom
    mpm = prob * (1.0 - m)
    sel = m == 0
    x = jnp.where(sel, mpm, 0.0)
    sparsity = _BUDGET / _W
    xbar = jnp.mean(x, axis=1, keepdims=True)
    r = sparsity / xbar
    beta = (1.0 - sparsity) / (1.0 - xbar)
    le = (r <= 1.0).astype(x.dtype)
    normed = le * x * r + (1.0 - le) * (1.0 - (1.0 - x) * beta)
    mpm = jnp.where(sel, normed, mpm)
    hard = (mpm > u).astype(mpm.dtype)
    mpm_ref[...] = mpm
    rowval_ref[...] = m + hard


def _mask_kernel(ks_ref, rv_ref, mpm_ref, mask_ref,
                 out_ref, mask_out_ref, mask_new_ref, fpm_ref):
    c = pl.program_id(1)
    out_ref[...] = ks_ref[...] * rv_ref[...]

    @pl.when(c == 0)
    def _():
        mask_out_ref[...] = mask_ref[...]
        mask_new_ref[...] = jnp.broadcast_to(rv_ref[...], mask_new_ref.shape)
        fpm_ref[...] = jnp.broadcast_to(mpm_ref[...], fpm_ref.shape)


def kernel(mask, kspace, sampler):
    batch, coils, slc, height, width, _ = kspace.shape
    mask2d = mask[:, :, 0, 0, :, :].reshape(batch, width)
    u = jnp.asarray(_u_const((batch, width), kspace.dtype))

    mpm, rowval = pl.pallas_call(
        _policy_kernel,
        out_shape=[
            jax.ShapeDtypeStruct((batch, width), kspace.dtype),
            jax.ShapeDtypeStruct((batch, width), kspace.dtype),
        ],
    )(mask2d, sampler, u)

    ksv = jnp.swapaxes(kspace, 4, 5)
    maskv = jnp.swapaxes(mask, 4, 5)
    rv6 = rowval.reshape(batch, 1, 1, 1, 1, width)
    mpm6 = mpm.reshape(batch, 1, 1, 1, 1, width)
    grid = (batch, coils)
    row_spec = pl.BlockSpec((1, 1, 1, 1, 1, width), lambda b, c: (b, 0, 0, 0, 0, 0))
    big_spec = pl.BlockSpec((1, 1, slc, height, 2, width), lambda b, c: (b, c, 0, 0, 0, 0))
    mask_spec = pl.BlockSpec((1, 1, slc, height, 1, width), lambda b, c: (b, 0, 0, 0, 0, 0))
    bcast_spec = pl.BlockSpec((1, 1, 1, height, 1, width), lambda b, c: (b, 0, 0, 0, 0, 0))
    out6, mask_o, mask_new6, fpm6 = pl.pallas_call(
        _mask_kernel,
        grid=grid,
        in_specs=[big_spec, row_spec, row_spec, mask_spec],
        out_specs=[big_spec, mask_spec, bcast_spec, bcast_spec],
        out_shape=[
            jax.ShapeDtypeStruct((batch, coils, slc, height, 2, width), kspace.dtype),
            jax.ShapeDtypeStruct((batch, 1, slc, height, 1, width), mask.dtype),
            jax.ShapeDtypeStruct((batch, 1, 1, height, 1, width), kspace.dtype),
            jax.ShapeDtypeStruct((batch, 1, 1, height, 1, width), kspace.dtype),
        ],
    )(ksv, rv6, mpm6, maskv)
    masked_kspace = jnp.swapaxes(out6, 4, 5)
    mask_in = jnp.swapaxes(mask_o, 4, 5)
    mask_new = jnp.swapaxes(mask_new6, 4, 5)
    final_prob_mask = jnp.swapaxes(fpm6, 4, 5)
    return (masked_kspace, mask_in, mask_new, final_prob_mask)


# speedup vs baseline: 1.2037x; 1.0441x over previous
import jax
import jax.numpy as jnp
import numpy as np
from jax.experimental import pallas as pl
from jax.experimental.pallas import tpu as pltpu

_ACCELERATION = 4.0
_CENTER_FRACTION = 0.08
_W = 320
_SLOPE = 10.0
_NUM_ACTIONS = _W
_BUDGET = int(_NUM_ACTIONS / _ACCELERATION - _NUM_ACTIONS * _CENTER_FRACTION)

_U_CACHE = {}


def _u_const(shape, dtype):
    k = (shape, np.dtype(dtype).name)
    if k not in _U_CACHE:
        with jax.ensure_compile_time_eval():
            _U_CACHE[k] = np.asarray(
                jax.random.uniform(jax.random.key(1), shape, dtype=dtype))
    return _U_CACHE[k]


def _policy_rows(m, s, u):
    prob = jnp.logaddexp(0.0, _SLOPE * s) / _SLOPE
    denom = jnp.max((1.0 - m) * prob, axis=1, keepdims=True)
    prob = prob / denom
    mpm = prob * (1.0 - m)
    sel = m == 0
    x = jnp.where(sel, mpm, 0.0)
    sparsity = _BUDGET / _W
    xbar = jnp.mean(x, axis=1, keepdims=True)
    r = sparsity / xbar
    beta = (1.0 - sparsity) / (1.0 - xbar)
    le = (r <= 1.0).astype(x.dtype)
    normed = le * x * r + (1.0 - le) * (1.0 - (1.0 - x) * beta)
    mpm = jnp.where(sel, normed, mpm)
    hard = (mpm > u).astype(mpm.dtype)
    return mpm, m + hard


def _fused_kernel(maskrow_ref, sampler_ref, u_ref, ks_ref, mask_ref,
                  out_ref, mask_out_ref, mask_new_ref, fpm_ref,
                  mpm_s, rv_s):
    c = pl.program_id(1)

    @pl.when(c == 0)
    def _():
        m = maskrow_ref[...].reshape(1, _W)
        mpm, rowval = _policy_rows(m, sampler_ref[...], u_ref[...].reshape(1, _W))
        mpm_s[...] = mpm
        rv_s[...] = rowval
        mask_out_ref[...] = mask_ref[...]
        mask_new_ref[...] = jnp.broadcast_to(
            rowval.reshape(1, 1, 1, 1, 1, _W), mask_new_ref.shape)
        fpm_ref[...] = jnp.broadcast_to(
            mpm.reshape(1, 1, 1, 1, 1, _W), fpm_ref.shape)

    out_ref[...] = ks_ref[...] * rv_s[...].reshape(1, 1, 1, 1, 1, _W)


def kernel(mask, kspace, sampler):
    batch, coils, slc, height, width, _ = kspace.shape
    u3 = jnp.asarray(_u_const((batch, width), kspace.dtype)).reshape(batch, 1, width)

    ksv = jnp.swapaxes(kspace, 4, 5)
    maskv = jnp.swapaxes(mask, 4, 5)
    grid = (batch, coils)
    row_spec = pl.BlockSpec((1, 1, 1, 1, 1, width), lambda b, c: (b, 0, 0, 0, 0, 0))
    big_spec = pl.BlockSpec((1, 1, slc, height, 2, width), lambda b, c: (b, c, 0, 0, 0, 0))
    mask_spec = pl.BlockSpec((1, 1, slc, height, 1, width), lambda b, c: (b, 0, 0, 0, 0, 0))
    bcast_spec = pl.BlockSpec((1, 1, 1, height, 1, width), lambda b, c: (b, 0, 0, 0, 0, 0))
    out6, mask_o, mask_new6, fpm6 = pl.pallas_call(
        _fused_kernel,
        grid=grid,
        in_specs=[
            row_spec,
            pl.BlockSpec((1, width), lambda b, c: (0, 0)),
            pl.BlockSpec((1, 1, width), lambda b, c: (b, 0, 0)),
            big_spec,
            mask_spec,
        ],
        out_specs=[big_spec, mask_spec, bcast_spec, bcast_spec],
        out_shape=[
            jax.ShapeDtypeStruct((batch, coils, slc, height, 2, width), kspace.dtype),
            jax.ShapeDtypeStruct((batch, 1, slc, height, 1, width), mask.dtype),
            jax.ShapeDtypeStruct((batch, 1, 1, height, 1, width), kspace.dtype),
            jax.ShapeDtypeStruct((batch, 1, 1, height, 1, width), kspace.dtype),
        ],
        scratch_shapes=[
            pltpu.VMEM((1, width), kspace.dtype),
            pltpu.VMEM((1, width), kspace.dtype),
        ],
    )(maskv, sampler, u3, ksv, maskv)
    masked_kspace = jnp.swapaxes(out6, 4, 5)
    mask_in = jnp.swapaxes(mask_o, 4, 5)
    mask_new = jnp.swapaxes(mask_new6, 4, 5)
    final_prob_mask = jnp.swapaxes(fpm6, 4, 5)
    return (masked_kspace, mask_in, mask_new, final_prob_mask)
